# baseline (speedup 1.0000x reference)
import functools

import jax
import jax.numpy as jnp
from jax import lax
from jax.experimental import pallas as pl
from jax.experimental.pallas import tpu as pltpu
from jax.experimental.pallas import tpu_sc as plsc

_BATCH = 16384
_D = 16
_L = 16
_NC = 2
_NS = 16
_NW = _NC * _NS
_BPW = _BATCH // _NW
_CHUNK = 128
_NCHUNK = _BPW // _CHUNK
_EPS = 1e-5
_LN2 = 0.6931471805599453


def _shr(i, n):
    return lax.shift_right_logical(i, jnp.full(i.shape, n, jnp.int32))


def _sqrt16(x):
    i = lax.bitcast_convert_type(x, jnp.int32)
    y = lax.bitcast_convert_type(jnp.int32(0x5F3759DF) - _shr(i, 1),
                                 jnp.float32)
    for _ in range(3):
        y = y * (1.5 - 0.5 * x * y * y)
    return x * y


def _ln16(z):
    i = lax.bitcast_convert_type(z, jnp.int32)
    e = _shr(i, 23) - 127
    m = lax.bitcast_convert_type((i & 0x007FFFFF) | 0x3F800000, jnp.float32)
    big = m > 1.4142135623730951
    m = jnp.where(big, m * 0.5, m)
    ef = (e + jnp.where(big, 1, 0)).astype(jnp.float32)
    s = (m - 1.0) / (m + 1.0)
    s2 = s * s
    p = 2.0 + s2 * (0.66666666666 + s2 * (0.4 + s2 * 0.28571428571))
    return ef * _LN2 + s * p


def _group(rows_u, rows_v, g, r16, t16):
    ri = lax.iota(jnp.int32, _L) + g * _L
    su = jnp.zeros((_L,), jnp.float32)
    sv = jnp.zeros((_L,), jnp.float32)
    sd = jnp.zeros((_L,), jnp.float32)
    for j in range(_D):
        cj = jnp.full((_L,), j, jnp.int32)
        cu = plsc.load_gather(rows_u, [ri, cj])
        cv = plsc.load_gather(rows_v, [ri, cj])
        su = su + cu * cu
        sv = sv + cv * cv
        d = cu - cv
        sd = sd + d * d
    omu = 1.0 - jnp.clip(su, 0.0, 1.0 - _EPS)
    omv = 1.0 - jnp.clip(sv, 0.0, 1.0 - _EPS)
    q = 2.0 * _sqrt16(sd + _EPS) / (omu * omv)
    duv = _ln16(1.0 + q + _sqrt16(q * (q + 2.0)))
    return 1.0 / (jnp.exp((duv - r16) / t16) + 1.0)


def _body(u_hbm, v_hbm, theta_hbm, r_hbm, t_hbm, out_hbm,
          idx_u, idx_v, rows_u, rows_v, out_v, r_v, t_v, sem):
    cid = lax.axis_index("c")
    sid = lax.axis_index("s")
    wid = sid * _NC + cid
    base = wid * _BPW
    pltpu.sync_copy(r_hbm, r_v)
    pltpu.sync_copy(t_hbm, t_v)
    for c in range(_NCHUNK):
        pltpu.sync_copy(u_hbm.at[pl.ds(base + c * _CHUNK, _CHUNK)],
                        idx_u.at[c])
        pltpu.sync_copy(v_hbm.at[pl.ds(base + c * _CHUNK, _CHUNK)],
                        idx_v.at[c])
    cps = []
    for c in range(_NCHUNK):
        dst_u = rows_u.at[pl.ds(c * _CHUNK, _CHUNK)]
        dst_v = rows_v.at[pl.ds(c * _CHUNK, _CHUNK)]
        cps.append(pltpu.async_copy(theta_hbm.at[idx_u.at[c]], dst_u, sem))
        cps.append(pltpu.async_copy(theta_hbm.at[idx_v.at[c]], dst_v, sem))
    for cp in cps:
        cp.wait()

    def gbody(g, carry):
        res = _group(rows_u, rows_v, g, r_v[...], t_v[...])
        out_v[pl.ds(g * _L, _L)] = res
        return carry

    lax.fori_loop(0, _BPW // _L, gbody, 0)
    pltpu.sync_copy(out_v, out_hbm.at[pl.ds(base, _BPW)])


@functools.cache
def _poincare_sc():
    mesh = plsc.VectorSubcoreMesh(core_axis_name="c", subcore_axis_name="s",
                                  num_cores=_NC, num_subcores=_NS)
    return pl.kernel(
        _body,
        out_type=jax.ShapeDtypeStruct((_BATCH,), jnp.float32),
        mesh=mesh,
        scratch_types=[
            pltpu.VMEM((_NCHUNK, _CHUNK), jnp.int32),
            pltpu.VMEM((_NCHUNK, _CHUNK), jnp.int32),
            pltpu.VMEM((_BPW, _D), jnp.float32),
            pltpu.VMEM((_BPW, _D), jnp.float32),
            pltpu.VMEM((_BPW,), jnp.float32),
            pltpu.VMEM((_L,), jnp.float32),
            pltpu.VMEM((_L,), jnp.float32),
            pltpu.SemaphoreType.DMA,
        ],
        compiler_params=pltpu.CompilerParams(needs_layout_passes=False,
                                             use_tc_tiling_on_sc=False),
    )


def kernel(u, v, theta, r, t):
    r16 = jnp.broadcast_to(jnp.reshape(r, (1,)).astype(jnp.float32), (_L,))
    t16 = jnp.broadcast_to(jnp.reshape(t, (1,)).astype(jnp.float32), (_L,))
    return _poincare_sc()(u.astype(jnp.int32), v.astype(jnp.int32),
                          theta, r16, t16)
